# Initial kernel scaffold; baseline (speedup 1.0000x reference)
import jax
import jax.numpy as jnp
from jax.experimental import pallas as pl


def kernel(x, w1, bn1_gamma, bn1_beta, bn1_mean, bn1_var, w2, bn2_gamma, bn2_beta, bn2_mean, bn2_var, w_sc, bn3_gamma, bn3_beta, bn3_mean, bn3_var):
    raise NotImplementedError("write your pallas kernel here")


# speedup vs baseline: 2.7138x; 2.7138x over previous
from functools import partial

import jax
import jax.numpy as jnp
from jax.experimental import pallas as pl
from jax.experimental.pallas import tpu as pltpu


def _fold_bn(gamma, beta, mean, var, eps=1e-5):
    scale = gamma / jnp.sqrt(var + eps)
    return scale, beta - mean * scale


def _band(w_dhwio, scale, wo, pad):
    k = w_dhwio.shape[0]
    ci, co = w_dhwio.shape[3], w_dhwio.shape[4]
    w = w_dhwio * scale
    w_in = jnp.arange(wo)[:, None]
    w_out = jnp.arange(wo)[None, :]
    kw = w_in - w_out + pad
    valid = (kw >= 0) & (kw < k)
    g = jnp.take(w, jnp.clip(kw, 0, k - 1), axis=2)
    g = jnp.where(valid[None, None, :, :, None, None], g, 0.0)
    g = jnp.transpose(g, (0, 1, 2, 4, 3, 5))
    return g.reshape(k * k, wo * ci, wo * co).astype(jnp.bfloat16)


def _fused_block_kernel(xp_ref, w1_ref, wsc_ref, w2_ref, b1_ref, b2_ref,
                        b3_ref, y_ref, h_scr, sc_scr, *, do, ho, kin, lanes):
    rows = do * ho

    acc = jnp.zeros((rows, lanes), jnp.float32)
    t = 0
    for kd in range(5):
        for kh in range(5):
            slab = xp_ref[pl.ds(kd, do), pl.ds(kh, ho), :].reshape(rows, kin)
            acc = acc + jnp.dot(slab, wsc_ref[t],
                                preferred_element_type=jnp.float32)
            t += 1
    sc_scr[...] = acc + b3_ref[...]

    acc = jnp.zeros((rows, lanes), jnp.float32)
    t = 0
    for kd in range(3):
        for kh in range(3):
            slab = xp_ref[pl.ds(kd + 1, do), pl.ds(kh + 1, ho), :]
            acc = acc + jnp.dot(slab.reshape(rows, kin), w1_ref[t],
                                preferred_element_type=jnp.float32)
            t += 1
    h = jnp.maximum(acc + b1_ref[...], 0.0).astype(jnp.bfloat16)

    h_scr[...] = jnp.zeros((do + 2, ho + 2, lanes), jnp.bfloat16)
    h_scr[pl.ds(1, do), pl.ds(1, ho), :] = h.reshape(do, ho, lanes)

    acc = jnp.zeros((rows, lanes), jnp.float32)
    t = 0
    for kd in range(3):
        for kh in range(3):
            slab = h_scr[pl.ds(kd, do), pl.ds(kh, ho), :].reshape(rows, lanes)
            acc = acc + jnp.dot(slab, w2_ref[t],
                                preferred_element_type=jnp.float32)
            t += 1
    y_ref[...] = jnp.maximum(acc + b2_ref[...] + sc_scr[...], 0.0)


def kernel(x, w1, bn1_gamma, bn1_beta, bn1_mean, bn1_var,
           w2, bn2_gamma, bn2_beta, bn2_mean, bn2_var,
           w_sc, bn3_gamma, bn3_beta, bn3_mean, bn3_var):
    B, Cin, D, H, W = x.shape
    Cout = w1.shape[-1]
    Do, Ho, Wo = D, H, W
    kin = W * Cin
    lanes = Wo * Cout
    rows = Do * Ho

    x_cl = jnp.transpose(x.astype(jnp.bfloat16), (0, 2, 3, 4, 1))
    x_cl = x_cl.reshape(B, D, H, kin)
    xp = jnp.pad(x_cl, ((0, 0), (2, 2), (2, 2), (0, 0)))
    Dp, Hp = D + 4, H + 4

    s1, c1 = _fold_bn(bn1_gamma, bn1_beta, bn1_mean, bn1_var)
    s2, c2 = _fold_bn(bn2_gamma, bn2_beta, bn2_mean, bn2_var)
    s3, c3 = _fold_bn(bn3_gamma, bn3_beta, bn3_mean, bn3_var)

    w1b = _band(w1, s1, Wo, 1)
    wscb = _band(w_sc, s3, Wo, 2)
    w2b = _band(w2, s2, Wo, 1)
    b1t = jnp.tile(c1, Wo).reshape(1, lanes).astype(jnp.float32)
    b2t = jnp.tile(c2, Wo).reshape(1, lanes).astype(jnp.float32)
    b3t = jnp.tile(c3, Wo).reshape(1, lanes).astype(jnp.float32)

    kern = partial(_fused_block_kernel, do=Do, ho=Ho, kin=kin, lanes=lanes)
    flops = 2 * B * rows * (kin * 34 + lanes * 9) * lanes
    bytes_accessed = int(xp.size * 2 + (w1b.size + wscb.size + w2b.size) * 2
                         + B * rows * lanes * 4)

    y = pl.pallas_call(
        kern,
        out_shape=jax.ShapeDtypeStruct((B, rows, lanes), jnp.float32),
        grid=(B,),
        in_specs=[
            pl.BlockSpec((None, Dp, Hp, kin), lambda b: (b, 0, 0, 0)),
            pl.BlockSpec((9, kin, lanes), lambda b: (0, 0, 0)),
            pl.BlockSpec((25, kin, lanes), lambda b: (0, 0, 0)),
            pl.BlockSpec((9, lanes, lanes), lambda b: (0, 0, 0)),
            pl.BlockSpec((1, lanes), lambda b: (0, 0)),
            pl.BlockSpec((1, lanes), lambda b: (0, 0)),
            pl.BlockSpec((1, lanes), lambda b: (0, 0)),
        ],
        out_specs=pl.BlockSpec((None, rows, lanes), lambda b: (b, 0, 0)),
        scratch_shapes=[
            pltpu.VMEM((Do + 2, Ho + 2, lanes), jnp.bfloat16),
            pltpu.VMEM((rows, lanes), jnp.float32),
        ],
        compiler_params=pltpu.CompilerParams(
            dimension_semantics=("parallel",),
            vmem_limit_bytes=64 * 1024 * 1024,
        ),
        cost_estimate=pl.CostEstimate(flops=flops, transcendentals=0,
                                      bytes_accessed=bytes_accessed),
    )(xp, w1b, wscb, w2b, b1t, b2t, b3t)

    y = y.reshape(B, Do, Ho, Wo, Cout)
    return jnp.transpose(y, (0, 4, 1, 2, 3))
